# baseline (speedup 1.0000x reference)
import functools

import jax
import jax.numpy as jnp
from jax import lax
from jax.experimental import pallas as pl
from jax.experimental.pallas import tpu as pltpu
from jax.experimental.pallas import tpu_sc as plsc

_NC, _NS = 2, 16
_NW = _NC * _NS
_NBUF = 2


def _sc_gather_planes(img3, widx, n_planes, H, W):
    per_w = n_planes // _NW
    mesh = plsc.VectorSubcoreMesh(core_axis_name="c", subcore_axis_name="s")

    @functools.partial(
        pl.kernel,
        out_type=jax.ShapeDtypeStruct((n_planes, H, W), jnp.float32),
        mesh=mesh,
        scratch_types=(
            [pltpu.VMEM((widx.shape[1],), jnp.int32)]
            + [pltpu.VMEM((1, H, W), jnp.float32) for _ in range(_NBUF)]
            + [pltpu.SemaphoreType.DMA((_NBUF,)),
               pltpu.SemaphoreType.DMA((_NBUF,))]
        ),
    )
    def k(img_hbm, widx_hbm, out_hbm, iv, *rest):
        bufs, gsem, ssem = rest[:_NBUF], rest[_NBUF], rest[_NBUF + 1]
        wid = lax.axis_index("s") * _NC + lax.axis_index("c")
        pltpu.sync_copy(widx_hbm.at[wid], iv)
        vec = iv[...]

        def start_gather(i, slot):
            c = pltpu.make_async_copy(
                img_hbm.at[pl.ds(vec[i], 1)], bufs[slot], gsem.at[slot])
            c.start()
            return c

        def start_scatter(i, slot):
            c = pltpu.make_async_copy(
                bufs[slot], out_hbm.at[pl.ds(wid * per_w + i, 1)],
                ssem.at[slot])
            c.start()
            return c

        gathers = [None] * per_w
        scatters = [None] * per_w
        for t in range(min(_NBUF, per_w)):
            gathers[t] = start_gather(t, t)
        for t in range(per_w):
            slot = t % _NBUF
            gathers[t].wait()
            scatters[t] = start_scatter(t, slot)
            if t + _NBUF < per_w:
                scatters[t].wait()
                gathers[t + _NBUF] = start_gather(t + _NBUF, slot)
        for t in range(max(0, per_w - _NBUF), per_w):
            scatters[t].wait()

    return k(img3, widx)


def _worker_plane_rows(indices, B, V, C, n):
    base = (jnp.arange(B, dtype=jnp.int32)[:, None] * V + indices) * C
    rows = base[:, :, None] + jnp.arange(C, dtype=jnp.int32)[None, None, :]
    flat = rows.reshape(_NW, -1)
    per_w = flat.shape[1]
    pad = (-per_w) % 16
    return jnp.pad(flat, ((0, 0), (0, pad)))


def _tc_copy_body(idx_ref, in_ref, out_ref):
    out_ref[...] = in_ref[...]


def _tc_gather_views(image, indices, n):
    B, V, C, H, W = image.shape
    return pl.pallas_call(
        _tc_copy_body,
        grid_spec=pltpu.PrefetchScalarGridSpec(
            num_scalar_prefetch=1,
            grid=(B, n, C),
            in_specs=[pl.BlockSpec((1, 1, 1, H, W),
                                   lambda b, v, c, idx: (b, idx[b, v], c, 0, 0))],
            out_specs=pl.BlockSpec((1, 1, 1, H, W),
                                   lambda b, v, c, idx: (b, v, c, 0, 0)),
        ),
        out_shape=jax.ShapeDtypeStruct((B, n, C, H, W), image.dtype),
    )(indices, image)


def kernel(image, context_indices, target_indices):
    B, V, C, H, W = image.shape
    n_in = context_indices.shape[1]
    n_tg = target_indices.shape[1]
    img3 = image.reshape(B * V * C, H, W)
    widx_in = _worker_plane_rows(context_indices, B, V, C, n_in)
    out_in = _sc_gather_planes(img3, widx_in, B * n_in * C, H, W)
    input_image = out_in.reshape(B, n_in, C, H, W)
    target_image = _tc_gather_views(image, target_indices, n_tg)
    return (input_image, target_image, context_indices, target_indices)


# speedup vs baseline: 1.5052x; 1.5052x over previous
import functools

import jax
import jax.numpy as jnp
from jax import lax
from jax.experimental import pallas as pl
from jax.experimental.pallas import tpu as pltpu
from jax.experimental.pallas import tpu_sc as plsc

_NC, _NS = 2, 16
_NW = _NC * _NS
_NBUF = 2


def _sc_gather_planes(img3, widx, n_planes, H, W):
    per_w = n_planes // _NW
    mesh = plsc.VectorSubcoreMesh(core_axis_name="c", subcore_axis_name="s")

    @functools.partial(
        pl.kernel,
        out_type=jax.ShapeDtypeStruct((n_planes, H, W), jnp.float32),
        mesh=mesh,
        scratch_types=(
            [pltpu.VMEM((widx.shape[1],), jnp.int32)]
            + [pltpu.VMEM((1, H, W), jnp.float32) for _ in range(_NBUF)]
            + [pltpu.SemaphoreType.DMA((_NBUF,)),
               pltpu.SemaphoreType.DMA((_NBUF,))]
        ),
    )
    def k(img_hbm, widx_hbm, out_hbm, iv, *rest):
        bufs, gsem, ssem = rest[:_NBUF], rest[_NBUF], rest[_NBUF + 1]
        wid = lax.axis_index("s") * _NC + lax.axis_index("c")
        pltpu.sync_copy(widx_hbm.at[wid], iv)
        vec = iv[...]

        def start_gather(i, slot):
            c = pltpu.make_async_copy(
                img_hbm.at[pl.ds(vec[i], 1)], bufs[slot], gsem.at[slot])
            c.start()
            return c

        def start_scatter(i, slot):
            c = pltpu.make_async_copy(
                bufs[slot], out_hbm.at[pl.ds(wid * per_w + i, 1)],
                ssem.at[slot])
            c.start()
            return c

        gathers = [None] * per_w
        scatters = [None] * per_w
        for t in range(min(_NBUF, per_w)):
            gathers[t] = start_gather(t, t)
        for t in range(per_w):
            slot = t % _NBUF
            gathers[t].wait()
            scatters[t] = start_scatter(t, slot)
            if t + _NBUF < per_w:
                scatters[t].wait()
                gathers[t + _NBUF] = start_gather(t + _NBUF, slot)
        for t in range(max(0, per_w - _NBUF), per_w):
            scatters[t].wait()

    return k(img3, widx)


def _worker_plane_rows(indices, B, V, C, n):
    base = (jnp.arange(B, dtype=jnp.int32)[:, None] * V + indices) * C
    rows = base[:, :, None] + jnp.arange(C, dtype=jnp.int32)[None, None, :]
    flat = rows.reshape(_NW, -1)
    per_w = flat.shape[1]
    pad = (-per_w) % 16
    return jnp.pad(flat, ((0, 0), (0, pad)))


def _tc_copy_body(idx_ref, in_ref, out_ref):
    out_ref[...] = in_ref[...]


def _tc_gather_views(image, indices, n):
    B, V, C, H, W = image.shape
    return pl.pallas_call(
        _tc_copy_body,
        grid_spec=pltpu.PrefetchScalarGridSpec(
            num_scalar_prefetch=1,
            grid=(B, n),
            in_specs=[pl.BlockSpec((1, 1, C, H, W),
                                   lambda b, v, idx: (b, idx[b, v], 0, 0, 0))],
            out_specs=pl.BlockSpec((1, 1, C, H, W),
                                   lambda b, v, idx: (b, v, 0, 0, 0)),
        ),
        out_shape=jax.ShapeDtypeStruct((B, n, C, H, W), image.dtype),
    )(indices, image)


def kernel(image, context_indices, target_indices):
    B, V, C, H, W = image.shape
    n_in = context_indices.shape[1]
    n_tg = target_indices.shape[1]
    img3 = image.reshape(B * V * C, H, W)
    widx_in = _worker_plane_rows(context_indices, B, V, C, n_in)
    out_in = _sc_gather_planes(img3, widx_in, B * n_in * C, H, W)
    input_image = out_in.reshape(B, n_in, C, H, W)
    target_image = _tc_gather_views(image, target_indices, n_tg)
    return (input_image, target_image, context_indices, target_indices)
